# baseline (speedup 1.0000x reference)
import dataclasses
import functools

import numpy as np
import jax
import jax.numpy as jnp
from jax import lax
from jax.experimental import pallas as pl
from jax.experimental.pallas import tpu as pltpu
from jax.experimental.pallas import tpu_sc as plsc

_RADIUS2 = np.float32(0.2 * 0.2)
_K = 32
_L = 16
_NC = 2
_NS = 16


def _qag(xyz, nxyz, features):
    B = xyz.shape[0]
    N = xyz.shape[1] // 3
    S = nxyz.shape[1] // 3
    C = features.shape[1]
    K = _K
    NW = _NC * _NS
    QW = (B * S) // NW
    WPB = NW // B
    CW = C // WPB
    QTR = (S * K) // 4
    RCH = 1024

    mesh = plsc.VectorSubcoreMesh(core_axis_name="c", subcore_axis_name="s")
    cp = pltpu.CompilerParams()
    if "needs_layout_passes" in pltpu.CompilerParams.__dataclass_fields__:
        cp = dataclasses.replace(cp, needs_layout_passes=False)

    @functools.partial(
        pl.kernel,
        out_type=jax.ShapeDtypeStruct((B, 3 + C, S * K), jnp.float32),
        mesh=mesh,
        compiler_params=cp,
        scratch_types=[
            pltpu.VMEM((S * K,), jnp.int32),
            pltpu.VMEM_SHARED((2, S * K), jnp.int32),
            pltpu.SemaphoreType.DMA,
            pltpu.SemaphoreType.DMA,
            pltpu.SemaphoreType.DMA,
            pltpu.SemaphoreType.DMA,
        ],
    )
    def qag(xyz_hbm, nxyz_hbm, feat_hbm, out_hbm,
            idx_all, shidx, rs0, rs1, os0, os1):
        iota16 = lax.iota(jnp.int32, _L)

        def _splat(v, lane):
            sel = jnp.where(iota16 == lane, v, jnp.zeros_like(v))
            return jnp.full((_L,), jnp.sum(sel), dtype=v.dtype)

        wid = lax.axis_index("c") * _NS + lax.axis_index("s")
        b = wid // WPB
        slot = b % 2
        qoff = (wid % WPB) * QW

        def phase1(pts, q, idxbuf, raw, gxstage):
            @pl.loop(0, N, step=RCH)
            def _dein(p0):
                pltpu.sync_copy(xyz_hbm.at[b, pl.ds(p0 * 3, RCH * 3)], raw)

                @pl.loop(0, RCH, step=_L)
                def _lanes(i):
                    flat = (iota16 + i) * 3
                    for d in range(3):
                        pts[d, pl.ds(p0 + i, _L)] = plsc.load_gather(
                            raw, [flat + d])

            pltpu.sync_copy(nxyz_hbm.at[b, pl.ds(qoff * 3, QW * 3)],
                            raw.at[pl.ds(0, QW * 3)])

            @pl.loop(0, QW, step=_L)
            def _qlanes(i):
                flat = (iota16 + i) * 3
                for d in range(3):
                    q[d, pl.ds(i, _L)] = plsc.load_gather(raw, [flat + d])

            @pl.loop(0, QW)
            def _per_query(qi):
                g = (qi // _L) * _L
                lane = qi - g
                qx = _splat(q[0, pl.ds(g, _L)], lane)
                qy = _splat(q[1, pl.ds(g, _L)], lane)
                qz = _splat(q[2, pl.ds(g, _L)], lane)
                idxbuf[pl.ds(0, _L)] = jnp.zeros((_L,), jnp.int32)

                def cond(carry):
                    off, cnt = carry
                    return jnp.logical_and(cnt < K, off < N)

                def step(carry):
                    off, cnt = carry
                    xv = pts[0, pl.ds(off, _L)]
                    yv = pts[1, pl.ds(off, _L)]
                    zv = pts[2, pl.ds(off, _L)]
                    dx = qx - xv
                    dy = qy - yv
                    dz = qz - zv
                    d2 = dx * dx + dy * dy + dz * dz
                    m = d2 < _RADIUS2
                    plsc.store_compressed(idxbuf.at[pl.ds(cnt, _L)],
                                          iota16 + off, mask=m)
                    hits = jnp.sum(jnp.where(m, 1, 0))
                    return off + _L, cnt + hits

                _, cnt = lax.while_loop(cond, step,
                                        (jnp.int32(0), jnp.int32(0)))

                k0 = idxbuf[pl.ds(0, _L)]
                k1 = idxbuf[pl.ds(_L, _L)]
                first = _splat(k0, jnp.int32(0))
                cntv = jnp.full((_L,), cnt, jnp.int32)
                f0 = jnp.where(iota16 < cntv, k0, first)
                f1 = jnp.where(iota16 + _L < cntv, k1, first)
                idx_all[pl.ds((qoff + qi) * K, _L)] = f0
                idx_all[pl.ds((qoff + qi) * K + _L, _L)] = f1
                for d in range(3):
                    dv = jnp.full((_L,), d, jnp.int32)
                    g0 = plsc.load_gather(pts, [dv, f0])
                    g1 = plsc.load_gather(pts, [dv, f1])
                    qd = (qx, qy, qz)[d]
                    gxstage[pl.ds(d * QW * K + qi * K, _L)] = g0 - qd
                    gxstage[pl.ds(d * QW * K + qi * K + _L, _L)] = g1 - qd

            for d in range(3):
                pltpu.sync_copy(gxstage.at[pl.ds(d * QW * K, QW * K)],
                                out_hbm.at[b, d, pl.ds(qoff * K, QW * K)])
            pltpu.sync_copy(idx_all.at[pl.ds(qoff * K, QW * K)],
                            shidx.at[slot, pl.ds(qoff * K, QW * K)])

        pl.run_scoped(
            phase1,
            pltpu.VMEM((3, N), jnp.float32),
            pltpu.VMEM((3, QW), jnp.float32),
            pltpu.VMEM((3 * _L,), jnp.int32),
            pltpu.VMEM((RCH * 3,), jnp.float32),
            pltpu.VMEM((3 * QW * K,), jnp.float32),
        )

        plsc.subcore_barrier()

        coff = (wid % WPB) * CW
        rsems = [rs0, rs1]
        osems = [os0, os1]

        def phase2(rows, och):
            pltpu.sync_copy(shidx.at[slot], idx_all)

            def row_dma(ci, par):
                return pltpu.make_async_copy(
                    feat_hbm.at[b, coff + ci], rows.at[par], rsems[par])

            def out_dma(cc, quarter, qs):
                return pltpu.make_async_copy(
                    och.at[qs],
                    out_hbm.at[b, 3 + coff + cc,
                               pl.ds(quarter * QTR, QTR)],
                    osems[qs])

            row_dma(0, 0).start()
            row_dma(1, 1).start()

            @pl.loop(0, CW, step=2)
            def _cpair(ci):
                for par in range(2):
                    cc = ci + par
                    row_dma(cc, par).wait()
                    parv = jnp.full((_L,), par, jnp.int32)
                    for quarter in range(4):
                        qs = quarter % 2
                        gnum = cc * 4 + quarter

                        @pl.when(gnum >= 2)
                        def _wait_prev():
                            g2 = gnum - 2
                            out_dma(g2 // 4, g2 % 4, qs).wait()

                        @pl.loop(0, QTR, step=8 * _L)
                        def _gather(i):
                            for u in range(8):
                                o = i + u * _L
                                iv = idx_all[pl.ds(quarter * QTR + o, _L)]
                                och[qs, pl.ds(o, _L)] = plsc.load_gather(
                                    rows, [parv, iv])

                        out_dma(cc, quarter, qs).start()

                    @pl.when(cc + 2 < CW)
                    def _next_row():
                        row_dma(cc + 2, par).start()

            glast = CW * 4 - 1
            for g2 in (glast - 1, glast):
                out_dma(g2 // 4, g2 % 4, g2 % 2).wait()

        pl.run_scoped(
            phase2,
            pltpu.VMEM((2, N), jnp.float32),
            pltpu.VMEM((2, QTR), jnp.float32),
        )

    return qag(xyz, nxyz, features)


def kernel(xyz, new_xyz, features):
    B, N, _ = xyz.shape
    S = new_xyz.shape[1]
    C = features.shape[1]
    out = _qag(xyz.reshape(B, N * 3), new_xyz.reshape(B, S * 3), features)
    return out.reshape(B, 3 + C, S, _K)


# speedup vs baseline: 1.0015x; 1.0015x over previous
import dataclasses
import functools

import numpy as np
import jax
import jax.numpy as jnp
from jax import lax
from jax.experimental import pallas as pl
from jax.experimental.pallas import tpu as pltpu
from jax.experimental.pallas import tpu_sc as plsc

_RADIUS2 = np.float32(0.2 * 0.2)
_K = 32
_L = 16
_NC = 2
_NS = 16


def _qag(xyz, nxyz, features):
    B = xyz.shape[0]
    N = xyz.shape[1] // 3
    S = nxyz.shape[1] // 3
    C = features.shape[1]
    K = _K
    NW = _NC * _NS
    QW = (B * S) // NW
    WPB = NW // B
    CW = C // WPB
    QTR = (S * K) // 4
    RCH = 1024

    mesh = plsc.VectorSubcoreMesh(core_axis_name="c", subcore_axis_name="s")
    cp = pltpu.CompilerParams()
    if "needs_layout_passes" in pltpu.CompilerParams.__dataclass_fields__:
        cp = dataclasses.replace(cp, needs_layout_passes=False)

    @functools.partial(
        pl.kernel,
        out_type=jax.ShapeDtypeStruct((B, 3 + C, S * K), jnp.float32),
        mesh=mesh,
        compiler_params=cp,
        scratch_types=[
            pltpu.VMEM((S * K,), jnp.int32),
            pltpu.VMEM_SHARED((2, S * K), jnp.int32),
            pltpu.SemaphoreType.DMA,
            pltpu.SemaphoreType.DMA,
            pltpu.SemaphoreType.DMA,
            pltpu.SemaphoreType.DMA,
        ],
    )
    def qag(xyz_hbm, nxyz_hbm, feat_hbm, out_hbm,
            idx_all, shidx, rs0, rs1, os0, os1):
        iota16 = lax.iota(jnp.int32, _L)

        def _splat(v, lane):
            sel = jnp.where(iota16 == lane, v, jnp.zeros_like(v))
            return jnp.full((_L,), jnp.sum(sel), dtype=v.dtype)

        wid = lax.axis_index("c") * _NS + lax.axis_index("s")
        b = wid // WPB
        slot = b % 2
        qoff = (wid % WPB) * QW

        def phase1(pts, q, idxbuf, raw, gxstage):
          with jax.named_scope("p1_deinterleave"):
            @pl.loop(0, N, step=RCH)
            def _dein(p0):
                pltpu.sync_copy(xyz_hbm.at[b, pl.ds(p0 * 3, RCH * 3)], raw)

                @pl.loop(0, RCH, step=_L)
                def _lanes(i):
                    flat = (iota16 + i) * 3
                    for d in range(3):
                        pts[d, pl.ds(p0 + i, _L)] = plsc.load_gather(
                            raw, [flat + d])

            pltpu.sync_copy(nxyz_hbm.at[b, pl.ds(qoff * 3, QW * 3)],
                            raw.at[pl.ds(0, QW * 3)])

            @pl.loop(0, QW, step=_L)
            def _qlanes(i):
                flat = (iota16 + i) * 3
                for d in range(3):
                    q[d, pl.ds(i, _L)] = plsc.load_gather(raw, [flat + d])

          with jax.named_scope("p1_ballquery"):
            @pl.loop(0, QW)
            def _per_query(qi):
                g = (qi // _L) * _L
                lane = qi - g
                qx = _splat(q[0, pl.ds(g, _L)], lane)
                qy = _splat(q[1, pl.ds(g, _L)], lane)
                qz = _splat(q[2, pl.ds(g, _L)], lane)
                idxbuf[pl.ds(0, _L)] = jnp.zeros((_L,), jnp.int32)

                def cond(carry):
                    off, cnt = carry
                    return jnp.logical_and(cnt < K, off < N)

                def step(carry):
                    off, cnt = carry
                    xv = pts[0, pl.ds(off, _L)]
                    yv = pts[1, pl.ds(off, _L)]
                    zv = pts[2, pl.ds(off, _L)]
                    dx = qx - xv
                    dy = qy - yv
                    dz = qz - zv
                    d2 = dx * dx + dy * dy + dz * dz
                    m = d2 < _RADIUS2
                    plsc.store_compressed(idxbuf.at[pl.ds(cnt, _L)],
                                          iota16 + off, mask=m)
                    hits = jnp.sum(jnp.where(m, 1, 0))
                    return off + _L, cnt + hits

                _, cnt = lax.while_loop(cond, step,
                                        (jnp.int32(0), jnp.int32(0)))

                k0 = idxbuf[pl.ds(0, _L)]
                k1 = idxbuf[pl.ds(_L, _L)]
                first = _splat(k0, jnp.int32(0))
                cntv = jnp.full((_L,), cnt, jnp.int32)
                f0 = jnp.where(iota16 < cntv, k0, first)
                f1 = jnp.where(iota16 + _L < cntv, k1, first)
                idx_all[pl.ds((qoff + qi) * K, _L)] = f0
                idx_all[pl.ds((qoff + qi) * K + _L, _L)] = f1
                for d in range(3):
                    dv = jnp.full((_L,), d, jnp.int32)
                    g0 = plsc.load_gather(pts, [dv, f0])
                    g1 = plsc.load_gather(pts, [dv, f1])
                    qd = (qx, qy, qz)[d]
                    gxstage[pl.ds(d * QW * K + qi * K, _L)] = g0 - qd
                    gxstage[pl.ds(d * QW * K + qi * K + _L, _L)] = g1 - qd

          with jax.named_scope("p1_writeout"):
            for d in range(3):
                pltpu.sync_copy(gxstage.at[pl.ds(d * QW * K, QW * K)],
                                out_hbm.at[b, d, pl.ds(qoff * K, QW * K)])
            pltpu.sync_copy(idx_all.at[pl.ds(qoff * K, QW * K)],
                            shidx.at[slot, pl.ds(qoff * K, QW * K)])

        pl.run_scoped(
            phase1,
            pltpu.VMEM((3, N), jnp.float32),
            pltpu.VMEM((3, QW), jnp.float32),
            pltpu.VMEM((3 * _L,), jnp.int32),
            pltpu.VMEM((RCH * 3,), jnp.float32),
            pltpu.VMEM((3 * QW * K,), jnp.float32),
        )

        plsc.subcore_barrier()

        coff = (wid % WPB) * CW
        rsems = [rs0, rs1]
        osems = [os0, os1]

        def phase2(rows, och):
          with jax.named_scope("p2_group"):
            pltpu.sync_copy(shidx.at[slot], idx_all)

            def row_dma(ci, par):
                return pltpu.make_async_copy(
                    feat_hbm.at[b, coff + ci], rows.at[par], rsems[par])

            def out_dma(cc, quarter, qs):
                return pltpu.make_async_copy(
                    och.at[qs],
                    out_hbm.at[b, 3 + coff + cc,
                               pl.ds(quarter * QTR, QTR)],
                    osems[qs])

            row_dma(0, 0).start()
            row_dma(1, 1).start()

            @pl.loop(0, CW, step=2)
            def _cpair(ci):
                for par in range(2):
                    cc = ci + par
                    row_dma(cc, par).wait()
                    parv = jnp.full((_L,), par, jnp.int32)
                    for quarter in range(4):
                        qs = quarter % 2
                        gnum = cc * 4 + quarter

                        @pl.when(gnum >= 2)
                        def _wait_prev():
                            g2 = gnum - 2
                            out_dma(g2 // 4, g2 % 4, qs).wait()

                        @pl.loop(0, QTR, step=8 * _L)
                        def _gather(i):
                            for u in range(8):
                                o = i + u * _L
                                iv = idx_all[pl.ds(quarter * QTR + o, _L)]
                                och[qs, pl.ds(o, _L)] = plsc.load_gather(
                                    rows, [parv, iv])

                        out_dma(cc, quarter, qs).start()

                    @pl.when(cc + 2 < CW)
                    def _next_row():
                        row_dma(cc + 2, par).start()

            glast = CW * 4 - 1
            for g2 in (glast - 1, glast):
                out_dma(g2 // 4, g2 % 4, g2 % 2).wait()

        pl.run_scoped(
            phase2,
            pltpu.VMEM((2, N), jnp.float32),
            pltpu.VMEM((2, QTR), jnp.float32),
        )

    return qag(xyz, nxyz, features)


def kernel(xyz, new_xyz, features):
    B, N, _ = xyz.shape
    S = new_xyz.shape[1]
    C = features.shape[1]
    out = _qag(xyz.reshape(B, N * 3), new_xyz.reshape(B, S * 3), features)
    return out.reshape(B, 3 + C, S, _K)


# speedup vs baseline: 1.8774x; 1.8746x over previous
import dataclasses
import functools

import numpy as np
import jax
import jax.numpy as jnp
from jax import lax
from jax.experimental import pallas as pl
from jax.experimental.pallas import tpu as pltpu
from jax.experimental.pallas import tpu_sc as plsc

_RADIUS2 = np.float32(0.2 * 0.2)
_K = 32
_L = 16
_NC = 2
_NS = 16


def _qag(xyz, nxyz, features):
    B = xyz.shape[0]
    N = xyz.shape[1] // 3
    S = nxyz.shape[1] // 3
    C = features.shape[1]
    K = _K
    NW = _NC * _NS
    QW = (B * S) // NW
    WPB = NW // B
    CW = C // WPB
    QTR = (S * K) // 4
    RCH = 1024

    mesh = plsc.VectorSubcoreMesh(core_axis_name="c", subcore_axis_name="s")
    cp = pltpu.CompilerParams()
    if "needs_layout_passes" in pltpu.CompilerParams.__dataclass_fields__:
        cp = dataclasses.replace(cp, needs_layout_passes=False)

    @functools.partial(
        pl.kernel,
        out_type=jax.ShapeDtypeStruct((B, 3 + C, S * K), jnp.float32),
        mesh=mesh,
        compiler_params=cp,
        scratch_types=[
            pltpu.VMEM((S * K,), jnp.int32),
            pltpu.VMEM_SHARED((2, S * K), jnp.int32),
            pltpu.SemaphoreType.DMA,
            pltpu.SemaphoreType.DMA,
            pltpu.SemaphoreType.DMA,
            pltpu.SemaphoreType.DMA,
        ],
    )
    def qag(xyz_hbm, nxyz_hbm, feat_hbm, out_hbm,
            idx_all, shidx, rs0, rs1, os0, os1):
        iota16 = lax.iota(jnp.int32, _L)

        def _splat(v, lane):
            sel = jnp.where(iota16 == lane, v, jnp.zeros_like(v))
            return jnp.full((_L,), jnp.sum(sel), dtype=v.dtype)

        wid = lax.axis_index("c") * _NS + lax.axis_index("s")
        b = wid // WPB
        slot = b % 2
        qoff = (wid % WPB) * QW

        def phase1(pts, q, idxbuf, raw, gxstage):
          with jax.named_scope("p1_deinterleave"):
            @pl.loop(0, N, step=RCH)
            def _dein(p0):
                pltpu.sync_copy(xyz_hbm.at[b, pl.ds(p0 * 3, RCH * 3)], raw)

                @pl.loop(0, RCH, step=_L)
                def _lanes(i):
                    flat = (iota16 + i) * 3
                    for d in range(3):
                        pts[d, pl.ds(p0 + i, _L)] = plsc.load_gather(
                            raw, [flat + d])

            pltpu.sync_copy(nxyz_hbm.at[b, pl.ds(qoff * 3, QW * 3)],
                            raw.at[pl.ds(0, QW * 3)])

            @pl.loop(0, QW, step=_L)
            def _qlanes(i):
                flat = (iota16 + i) * 3
                for d in range(3):
                    q[d, pl.ds(i, _L)] = plsc.load_gather(raw, [flat + d])

          with jax.named_scope("p1_ballquery"):
            @pl.loop(0, QW)
            def _per_query(qi):
                g = (qi // _L) * _L
                lane = qi - g
                qx = _splat(q[0, pl.ds(g, _L)], lane)
                qy = _splat(q[1, pl.ds(g, _L)], lane)
                qz = _splat(q[2, pl.ds(g, _L)], lane)
                idxbuf[pl.ds(0, _L)] = jnp.zeros((_L,), jnp.int32)

                def cond(carry):
                    off, cnt = carry
                    return jnp.logical_and(cnt < K, off < N)

                def step(carry):
                    off, cnt = carry
                    xv = pts[0, pl.ds(off, _L)]
                    yv = pts[1, pl.ds(off, _L)]
                    zv = pts[2, pl.ds(off, _L)]
                    dx = qx - xv
                    dy = qy - yv
                    dz = qz - zv
                    d2 = dx * dx + dy * dy + dz * dz
                    m = d2 < _RADIUS2
                    plsc.store_compressed(idxbuf.at[pl.ds(cnt, _L)],
                                          iota16 + off, mask=m)
                    hits = jnp.sum(jnp.where(m, 1, 0))
                    return off + _L, cnt + hits

                _, cnt = lax.while_loop(cond, step,
                                        (jnp.int32(0), jnp.int32(0)))

                k0 = idxbuf[pl.ds(0, _L)]
                k1 = idxbuf[pl.ds(_L, _L)]
                first = _splat(k0, jnp.int32(0))
                cntv = jnp.full((_L,), cnt, jnp.int32)
                f0 = jnp.where(iota16 < cntv, k0, first)
                f1 = jnp.where(iota16 + _L < cntv, k1, first)
                idx_all[pl.ds((qoff + qi) * K, _L)] = f0
                idx_all[pl.ds((qoff + qi) * K + _L, _L)] = f1
                for d in range(3):
                    dv = jnp.full((_L,), d, jnp.int32)
                    g0 = plsc.load_gather(pts, [dv, f0])
                    g1 = plsc.load_gather(pts, [dv, f1])
                    qd = (qx, qy, qz)[d]
                    gxstage[pl.ds(d * QW * K + qi * K, _L)] = g0 - qd
                    gxstage[pl.ds(d * QW * K + qi * K + _L, _L)] = g1 - qd

          with jax.named_scope("p1_writeout"):
            for d in range(3):
                pltpu.sync_copy(gxstage.at[pl.ds(d * QW * K, QW * K)],
                                out_hbm.at[b, d, pl.ds(qoff * K, QW * K)])
            pltpu.sync_copy(idx_all.at[pl.ds(qoff * K, QW * K)],
                            shidx.at[slot, pl.ds(qoff * K, QW * K)])

        pl.run_scoped(
            phase1,
            pltpu.VMEM((3, N), jnp.float32),
            pltpu.VMEM((3, QW), jnp.float32),
            pltpu.VMEM((3 * _L,), jnp.int32),
            pltpu.VMEM((RCH * 3,), jnp.float32),
            pltpu.VMEM((3 * QW * K,), jnp.float32),
        )

        plsc.subcore_barrier()

        coff = (wid % WPB) * CW
        rsems = [rs0, rs1]
        osems = [os0, os1]

        def phase2(row0, row1, och0, och1):
          with jax.named_scope("p2_group"):
            pltpu.sync_copy(shidx.at[slot], idx_all)
            rowbufs = [row0, row1]
            ochbufs = [och0, och1]

            def row_dma(ci, par):
                return pltpu.make_async_copy(
                    feat_hbm.at[b, coff + ci], rowbufs[par], rsems[par])

            def out_dma(cc, quarter, qs):
                return pltpu.make_async_copy(
                    ochbufs[qs],
                    out_hbm.at[b, 3 + coff + cc,
                               pl.ds(quarter * QTR, QTR)],
                    osems[qs])

            row_dma(0, 0).start()
            row_dma(1, 1).start()

            @pl.loop(0, CW, step=2)
            def _cpair(ci):
                for par in range(2):
                    cc = ci + par
                    rowbuf = rowbufs[par]
                    row_dma(cc, par).wait()
                    for quarter in range(4):
                        qs = quarter % 2
                        ochbuf = ochbufs[qs]
                        gnum = cc * 4 + quarter

                        @pl.when(gnum >= 2)
                        def _wait_prev():
                            g2 = gnum - 2
                            out_dma(g2 // 4, g2 % 4, qs).wait()

                        @plsc.parallel_loop(0, QTR, step=_L, unroll=8)
                        def _gather(o):
                            iv = idx_all[pl.ds(quarter * QTR + o, _L)]
                            ochbuf[pl.ds(o, _L)] = plsc.load_gather(
                                rowbuf, [iv])

                        out_dma(cc, quarter, qs).start()

                    @pl.when(cc + 2 < CW)
                    def _next_row():
                        row_dma(cc + 2, par).start()

            glast = CW * 4 - 1
            for g2 in (glast - 1, glast):
                out_dma(g2 // 4, g2 % 4, g2 % 2).wait()

        pl.run_scoped(
            phase2,
            pltpu.VMEM((N,), jnp.float32),
            pltpu.VMEM((N,), jnp.float32),
            pltpu.VMEM((QTR,), jnp.float32),
            pltpu.VMEM((QTR,), jnp.float32),
        )

    return qag(xyz, nxyz, features)


def kernel(xyz, new_xyz, features):
    B, N, _ = xyz.shape
    S = new_xyz.shape[1]
    C = features.shape[1]
    out = _qag(xyz.reshape(B, N * 3), new_xyz.reshape(B, S * 3), features)
    return out.reshape(B, 3 + C, S, _K)


# speedup vs baseline: 2.1055x; 1.1215x over previous
import dataclasses
import functools

import numpy as np
import jax
import jax.numpy as jnp
from jax import lax
from jax.experimental import pallas as pl
from jax.experimental.pallas import tpu as pltpu
from jax.experimental.pallas import tpu_sc as plsc

_RADIUS2 = np.float32(0.2 * 0.2)
_K = 32
_L = 16
_NC = 2
_NS = 16


def _qag(xt, nxt, features):
    B, C, N = features.shape
    S = nxt.shape[0] // (3 * B)
    K = _K
    NW = _NC * _NS
    QW = (B * S) // NW
    WPB = NW // B
    CW = C // WPB
    QTR = (S * K) // 4

    mesh = plsc.VectorSubcoreMesh(core_axis_name="c", subcore_axis_name="s")
    cp = pltpu.CompilerParams()
    if "needs_layout_passes" in pltpu.CompilerParams.__dataclass_fields__:
        cp = dataclasses.replace(cp, needs_layout_passes=False)

    @functools.partial(
        pl.kernel,
        out_type=jax.ShapeDtypeStruct((B, 3 + C, S * K), jnp.float32),
        mesh=mesh,
        compiler_params=cp,
        scratch_types=[
            pltpu.VMEM((S * K,), jnp.int32),
            pltpu.VMEM_SHARED((2, S * K), jnp.int32),
            pltpu.SemaphoreType.DMA,
            pltpu.SemaphoreType.DMA,
            pltpu.SemaphoreType.DMA,
            pltpu.SemaphoreType.DMA,
        ],
    )
    def qag(xt_hbm, nxt_hbm, feat_hbm, out_hbm,
            idx_all, shidx, rs0, rs1, os0, os1):
        iota16 = lax.iota(jnp.int32, _L)

        def _splat(v, lane):
            sel = jnp.where(iota16 == lane, v, jnp.zeros_like(v))
            return jnp.full((_L,), jnp.sum(sel), dtype=v.dtype)

        wid = lax.axis_index("c") * _NS + lax.axis_index("s")
        b = wid // WPB
        slot = b % 2
        qoff = (wid % WPB) * QW

        def phase1(pts, q, idxbuf, gxstage):
          with jax.named_scope("p1_load"):
            for d in range(3):
                pltpu.sync_copy(xt_hbm.at[pl.ds(d * B * N + b * N, N)],
                                pts.at[pl.ds(d * N, N)])
                pltpu.sync_copy(
                    nxt_hbm.at[pl.ds(d * B * S + b * S + qoff, QW)],
                    q.at[pl.ds(d * QW, QW)])

          with jax.named_scope("p1_ballquery"):
            @pl.loop(0, QW)
            def _per_query(qi):
                g = (qi // _L) * _L
                lane = qi - g
                qx = _splat(q[pl.ds(g, _L)], lane)
                qy = _splat(q[pl.ds(QW + g, _L)], lane)
                qz = _splat(q[pl.ds(2 * QW + g, _L)], lane)
                idxbuf[pl.ds(0, _L)] = jnp.zeros((_L,), jnp.int32)

                def cond(carry):
                    off, cnt = carry
                    return jnp.logical_and(cnt < K, off < N)

                def step(carry):
                    off, cnt = carry
                    xv = pts[pl.ds(off, _L)]
                    yv = pts[pl.ds(N + off, _L)]
                    zv = pts[pl.ds(2 * N + off, _L)]
                    dx = qx - xv
                    dy = qy - yv
                    dz = qz - zv
                    d2 = dx * dx + dy * dy + dz * dz
                    m = d2 < _RADIUS2
                    plsc.store_compressed(idxbuf.at[pl.ds(cnt, _L)],
                                          iota16 + off, mask=m)
                    hits = jnp.sum(jnp.where(m, 1, 0))
                    return off + _L, cnt + hits

                _, cnt = lax.while_loop(cond, step,
                                        (jnp.int32(0), jnp.int32(0)))

                k0 = idxbuf[pl.ds(0, _L)]
                k1 = idxbuf[pl.ds(_L, _L)]
                first = _splat(k0, jnp.int32(0))
                cntv = jnp.full((_L,), cnt, jnp.int32)
                f0 = jnp.where(iota16 < cntv, k0, first)
                f1 = jnp.where(iota16 + _L < cntv, k1, first)
                idx_all[pl.ds((qoff + qi) * K, _L)] = f0
                idx_all[pl.ds((qoff + qi) * K + _L, _L)] = f1
                for d in range(3):
                    g0 = plsc.load_gather(pts, [f0 + d * N])
                    g1 = plsc.load_gather(pts, [f1 + d * N])
                    qd = (qx, qy, qz)[d]
                    gxstage[pl.ds(d * QW * K + qi * K, _L)] = g0 - qd
                    gxstage[pl.ds(d * QW * K + qi * K + _L, _L)] = g1 - qd

          with jax.named_scope("p1_writeout"):
            for d in range(3):
                pltpu.sync_copy(gxstage.at[pl.ds(d * QW * K, QW * K)],
                                out_hbm.at[b, d, pl.ds(qoff * K, QW * K)])
            pltpu.sync_copy(idx_all.at[pl.ds(qoff * K, QW * K)],
                            shidx.at[slot, pl.ds(qoff * K, QW * K)])

        pl.run_scoped(
            phase1,
            pltpu.VMEM((3 * N,), jnp.float32),
            pltpu.VMEM((3 * QW,), jnp.float32),
            pltpu.VMEM((3 * _L,), jnp.int32),
            pltpu.VMEM((3 * QW * K,), jnp.float32),
        )

        plsc.subcore_barrier()

        coff = (wid % WPB) * CW
        rsems = [rs0, rs1]
        osems = [os0, os1]

        def phase2(row0, row1, och0, och1):
          with jax.named_scope("p2_group"):
            pltpu.sync_copy(shidx.at[slot], idx_all)
            rowbufs = [row0, row1]
            ochbufs = [och0, och1]

            def row_dma(ci, par):
                return pltpu.make_async_copy(
                    feat_hbm.at[b, coff + ci], rowbufs[par], rsems[par])

            def out_dma(cc, quarter, qs):
                return pltpu.make_async_copy(
                    ochbufs[qs],
                    out_hbm.at[b, 3 + coff + cc,
                               pl.ds(quarter * QTR, QTR)],
                    osems[qs])

            row_dma(0, 0).start()
            row_dma(1, 1).start()

            @pl.loop(0, CW, step=2)
            def _cpair(ci):
                for par in range(2):
                    cc = ci + par
                    rowbuf = rowbufs[par]
                    row_dma(cc, par).wait()
                    for quarter in range(4):
                        qs = quarter % 2
                        ochbuf = ochbufs[qs]
                        gnum = cc * 4 + quarter

                        @pl.when(gnum >= 2)
                        def _wait_prev():
                            g2 = gnum - 2
                            out_dma(g2 // 4, g2 % 4, qs).wait()

                        @plsc.parallel_loop(0, QTR, step=_L, unroll=8)
                        def _gather(o):
                            iv = idx_all[pl.ds(quarter * QTR + o, _L)]
                            ochbuf[pl.ds(o, _L)] = plsc.load_gather(
                                rowbuf, [iv])

                        out_dma(cc, quarter, qs).start()

                    @pl.when(cc + 2 < CW)
                    def _next_row():
                        row_dma(cc + 2, par).start()

            glast = CW * 4 - 1
            for g2 in (glast - 1, glast):
                out_dma(g2 // 4, g2 % 4, g2 % 2).wait()

        pl.run_scoped(
            phase2,
            pltpu.VMEM((N,), jnp.float32),
            pltpu.VMEM((N,), jnp.float32),
            pltpu.VMEM((QTR,), jnp.float32),
            pltpu.VMEM((QTR,), jnp.float32),
        )

    return qag(xt, nxt, features)


def kernel(xyz, new_xyz, features):
    B, N, _ = xyz.shape
    S = new_xyz.shape[1]
    C = features.shape[1]
    xt = jnp.transpose(xyz, (2, 0, 1)).reshape(3 * B * N)
    nxt = jnp.transpose(new_xyz, (2, 0, 1)).reshape(3 * B * S)
    out = _qag(xt, nxt, features)
    return out.reshape(B, 3 + C, S, _K)


# speedup vs baseline: 3.2365x; 1.5372x over previous
import dataclasses
import functools

import numpy as np
import jax
import jax.numpy as jnp
from jax import lax
from jax.experimental import pallas as pl
from jax.experimental.pallas import tpu as pltpu
from jax.experimental.pallas import tpu_sc as plsc

_RADIUS2 = np.float32(0.2 * 0.2)
_K = 32
_L = 16
_NC = 2
_NS = 16


def _qag(xt, nxt, features):
    B, C, N = features.shape
    S = nxt.shape[0] // (3 * B)
    K = _K
    NW = _NC * _NS
    QW = (B * S) // NW
    WPB = NW // B
    CW = C // WPB
    QTR = (S * K) // 4

    mesh = plsc.VectorSubcoreMesh(core_axis_name="c", subcore_axis_name="s")
    cp = pltpu.CompilerParams()
    if "needs_layout_passes" in pltpu.CompilerParams.__dataclass_fields__:
        cp = dataclasses.replace(cp, needs_layout_passes=False)

    @functools.partial(
        pl.kernel,
        out_type=jax.ShapeDtypeStruct((B, 3 + C, S * K), jnp.float32),
        mesh=mesh,
        compiler_params=cp,
        scratch_types=[
            pltpu.VMEM((S * K,), jnp.int32),
            pltpu.VMEM_SHARED((2, S * K), jnp.int32),
            pltpu.SemaphoreType.DMA,
            pltpu.SemaphoreType.DMA,
            pltpu.SemaphoreType.DMA,
            pltpu.SemaphoreType.DMA,
        ],
    )
    def qag(xt_hbm, nxt_hbm, feat_hbm, out_hbm,
            idx_all, shidx, rs0, rs1, os0, os1):
        iota16 = lax.iota(jnp.int32, _L)

        def _splat(v, lane):
            sel = jnp.where(iota16 == lane, v, jnp.zeros_like(v))
            return jnp.full((_L,), jnp.sum(sel), dtype=v.dtype)

        wid = lax.axis_index("c") * _NS + lax.axis_index("s")
        b = wid // WPB
        slot = b % 2
        qoff = (wid % WPB) * QW

        def phase1(pts, q, idxbuf, gxstage):
          with jax.named_scope("p1_load"):
            for d in range(3):
                pltpu.sync_copy(xt_hbm.at[pl.ds(d * B * N + b * N, N)],
                                pts.at[pl.ds(d * N, N)])
                pltpu.sync_copy(
                    nxt_hbm.at[pl.ds(d * B * S + b * S + qoff, QW)],
                    q.at[pl.ds(d * QW, QW)])

          with jax.named_scope("p1_ballquery"):
            @pl.loop(0, QW)
            def _per_query(qi):
                g = (qi // _L) * _L
                lane = qi - g
                qx = _splat(q[pl.ds(g, _L)], lane)
                qy = _splat(q[pl.ds(QW + g, _L)], lane)
                qz = _splat(q[pl.ds(2 * QW + g, _L)], lane)
                idxbuf[pl.ds(0, _L)] = jnp.zeros((_L,), jnp.int32)

                def cond(carry):
                    off, cnt = carry
                    return jnp.logical_and(cnt < K, off < N)

                def step(carry):
                    off, cnt = carry
                    cntv = jnp.full((_L,), cnt, jnp.int32)
                    for u in range(8):
                        o = off + u * _L
                        xv = pts[pl.ds(o, _L)]
                        yv = pts[pl.ds(N + o, _L)]
                        zv = pts[pl.ds(2 * N + o, _L)]
                        dx = qx - xv
                        dy = qy - yv
                        dz = qz - zv
                        d2 = dx * dx + dy * dy + dz * dz
                        m = d2 < _RADIUS2
                        pfx = plsc.cumsum(jnp.where(m, 1, 0))
                        plsc.store_scatter(idxbuf, [cntv + (pfx - 1)],
                                           iota16 + o, mask=m)
                        cntv = cntv + plsc.all_reduce_population_count(m)
                    cnt = jnp.sum(jnp.where(iota16 == 0, cntv, 0))
                    return off + 8 * _L, cnt

                _, cnt = lax.while_loop(cond, step,
                                        (jnp.int32(0), jnp.int32(0)))

                k0 = idxbuf[pl.ds(0, _L)]
                k1 = idxbuf[pl.ds(_L, _L)]
                first = _splat(k0, jnp.int32(0))
                cntv = jnp.full((_L,), cnt, jnp.int32)
                f0 = jnp.where(iota16 < cntv, k0, first)
                f1 = jnp.where(iota16 + _L < cntv, k1, first)
                idx_all[pl.ds((qoff + qi) * K, _L)] = f0
                idx_all[pl.ds((qoff + qi) * K + _L, _L)] = f1
                for d in range(3):
                    g0 = plsc.load_gather(pts, [f0 + d * N])
                    g1 = plsc.load_gather(pts, [f1 + d * N])
                    qd = (qx, qy, qz)[d]
                    gxstage[pl.ds(d * QW * K + qi * K, _L)] = g0 - qd
                    gxstage[pl.ds(d * QW * K + qi * K + _L, _L)] = g1 - qd

          with jax.named_scope("p1_writeout"):
            for d in range(3):
                pltpu.sync_copy(gxstage.at[pl.ds(d * QW * K, QW * K)],
                                out_hbm.at[b, d, pl.ds(qoff * K, QW * K)])
            pltpu.sync_copy(idx_all.at[pl.ds(qoff * K, QW * K)],
                            shidx.at[slot, pl.ds(qoff * K, QW * K)])

        pl.run_scoped(
            phase1,
            pltpu.VMEM((3 * N,), jnp.float32),
            pltpu.VMEM((3 * QW,), jnp.float32),
            pltpu.VMEM((K + 8 * _L + _L,), jnp.int32),
            pltpu.VMEM((3 * QW * K,), jnp.float32),
        )

        plsc.subcore_barrier()

        coff = (wid % WPB) * CW
        rsems = [rs0, rs1]
        osems = [os0, os1]

        def phase2(row0, row1, och0, och1):
          with jax.named_scope("p2_group"):
            pltpu.sync_copy(shidx.at[slot], idx_all)
            rowbufs = [row0, row1]
            ochbufs = [och0, och1]

            def row_dma(ci, par):
                return pltpu.make_async_copy(
                    feat_hbm.at[b, coff + ci], rowbufs[par], rsems[par])

            def out_dma(cc, quarter, qs):
                return pltpu.make_async_copy(
                    ochbufs[qs],
                    out_hbm.at[b, 3 + coff + cc,
                               pl.ds(quarter * QTR, QTR)],
                    osems[qs])

            row_dma(0, 0).start()
            row_dma(1, 1).start()

            @pl.loop(0, CW, step=2)
            def _cpair(ci):
                for par in range(2):
                    cc = ci + par
                    rowbuf = rowbufs[par]
                    row_dma(cc, par).wait()
                    for quarter in range(4):
                        qs = quarter % 2
                        ochbuf = ochbufs[qs]
                        gnum = cc * 4 + quarter

                        @pl.when(gnum >= 2)
                        def _wait_prev():
                            g2 = gnum - 2
                            out_dma(g2 // 4, g2 % 4, qs).wait()

                        @plsc.parallel_loop(0, QTR, step=_L, unroll=8)
                        def _gather(o):
                            iv = idx_all[pl.ds(quarter * QTR + o, _L)]
                            ochbuf[pl.ds(o, _L)] = plsc.load_gather(
                                rowbuf, [iv])

                        out_dma(cc, quarter, qs).start()

                    @pl.when(cc + 2 < CW)
                    def _next_row():
                        row_dma(cc + 2, par).start()

            glast = CW * 4 - 1
            for g2 in (glast - 1, glast):
                out_dma(g2 // 4, g2 % 4, g2 % 2).wait()

        pl.run_scoped(
            phase2,
            pltpu.VMEM((N,), jnp.float32),
            pltpu.VMEM((N,), jnp.float32),
            pltpu.VMEM((QTR,), jnp.float32),
            pltpu.VMEM((QTR,), jnp.float32),
        )

    return qag(xt, nxt, features)


def kernel(xyz, new_xyz, features):
    B, N, _ = xyz.shape
    S = new_xyz.shape[1]
    C = features.shape[1]
    xt = jnp.transpose(xyz, (2, 0, 1)).reshape(3 * B * N)
    nxt = jnp.transpose(new_xyz, (2, 0, 1)).reshape(3 * B * S)
    out = _qag(xt, nxt, features)
    return out.reshape(B, 3 + C, S, _K)


# speedup vs baseline: 3.2805x; 1.0136x over previous
import dataclasses
import functools

import numpy as np
import jax
import jax.numpy as jnp
from jax import lax
from jax.experimental import pallas as pl
from jax.experimental.pallas import tpu as pltpu
from jax.experimental.pallas import tpu_sc as plsc

_RADIUS2 = np.float32(0.2 * 0.2)
_K = 32
_L = 16
_NC = 2
_NS = 16


def _qag(xt, nxt, features):
    B, C, N = features.shape
    S = nxt.shape[0] // (3 * B)
    K = _K
    NW = _NC * _NS
    QW = (B * S) // NW
    WPB = NW // B
    CW = C // WPB
    QTR = (S * K) // 4

    mesh = plsc.VectorSubcoreMesh(core_axis_name="c", subcore_axis_name="s")
    cp = pltpu.CompilerParams()
    if "needs_layout_passes" in pltpu.CompilerParams.__dataclass_fields__:
        cp = dataclasses.replace(cp, needs_layout_passes=False)

    @functools.partial(
        pl.kernel,
        out_type=jax.ShapeDtypeStruct((B, 3 + C, S * K), jnp.float32),
        mesh=mesh,
        compiler_params=cp,
        scratch_types=[
            pltpu.VMEM((S * K,), jnp.int32),
            pltpu.VMEM_SHARED((2, S * K), jnp.int32),
            pltpu.SemaphoreType.DMA,
            pltpu.SemaphoreType.DMA,
            pltpu.SemaphoreType.DMA,
            pltpu.SemaphoreType.DMA,
            pltpu.SemaphoreType.DMA,
            pltpu.SemaphoreType.DMA,
            pltpu.SemaphoreType.DMA,
            pltpu.SemaphoreType.DMA,
        ],
    )
    def qag(xt_hbm, nxt_hbm, feat_hbm, out_hbm,
            idx_all, shidx, rs0, rs1, rs2, rs3, os0, os1, os2, os3):
        iota16 = lax.iota(jnp.int32, _L)

        def _splat(v, lane):
            sel = jnp.where(iota16 == lane, v, jnp.zeros_like(v))
            return jnp.full((_L,), jnp.sum(sel), dtype=v.dtype)

        wid = lax.axis_index("c") * _NS + lax.axis_index("s")
        b = wid // WPB
        slot = b % 2
        qoff = (wid % WPB) * QW

        def phase1(pts, q, idxbuf, gxstage):
          with jax.named_scope("p1_load"):
            for d in range(3):
                pltpu.sync_copy(xt_hbm.at[pl.ds(d * B * N + b * N, N)],
                                pts.at[pl.ds(d * N, N)])
                pltpu.sync_copy(
                    nxt_hbm.at[pl.ds(d * B * S + b * S + qoff, QW)],
                    q.at[pl.ds(d * QW, QW)])

          with jax.named_scope("p1_ballquery"):
            @pl.loop(0, QW)
            def _per_query(qi):
                g = (qi // _L) * _L
                lane = qi - g
                qx = _splat(q[pl.ds(g, _L)], lane)
                qy = _splat(q[pl.ds(QW + g, _L)], lane)
                qz = _splat(q[pl.ds(2 * QW + g, _L)], lane)
                idxbuf[pl.ds(0, _L)] = jnp.zeros((_L,), jnp.int32)

                def cond(carry):
                    off, cnt = carry
                    return jnp.logical_and(cnt < K, off < N)

                def step(carry):
                    off, cnt = carry
                    cntv = jnp.full((_L,), cnt, jnp.int32)
                    for u in range(8):
                        o = off + u * _L
                        xv = pts[pl.ds(o, _L)]
                        yv = pts[pl.ds(N + o, _L)]
                        zv = pts[pl.ds(2 * N + o, _L)]
                        dx = qx - xv
                        dy = qy - yv
                        dz = qz - zv
                        d2 = dx * dx + dy * dy + dz * dz
                        m = d2 < _RADIUS2
                        pfx = plsc.cumsum(jnp.where(m, 1, 0))
                        plsc.store_scatter(idxbuf, [cntv + (pfx - 1)],
                                           iota16 + o, mask=m)
                        cntv = cntv + plsc.all_reduce_population_count(m)
                    cnt = jnp.sum(jnp.where(iota16 == 0, cntv, 0))
                    return off + 8 * _L, cnt

                _, cnt = lax.while_loop(cond, step,
                                        (jnp.int32(0), jnp.int32(0)))

                k0 = idxbuf[pl.ds(0, _L)]
                k1 = idxbuf[pl.ds(_L, _L)]
                first = _splat(k0, jnp.int32(0))
                cntv = jnp.full((_L,), cnt, jnp.int32)
                f0 = jnp.where(iota16 < cntv, k0, first)
                f1 = jnp.where(iota16 + _L < cntv, k1, first)
                idx_all[pl.ds((qoff + qi) * K, _L)] = f0
                idx_all[pl.ds((qoff + qi) * K + _L, _L)] = f1
                for d in range(3):
                    g0 = plsc.load_gather(pts, [f0 + d * N])
                    g1 = plsc.load_gather(pts, [f1 + d * N])
                    qd = (qx, qy, qz)[d]
                    gxstage[pl.ds(d * QW * K + qi * K, _L)] = g0 - qd
                    gxstage[pl.ds(d * QW * K + qi * K + _L, _L)] = g1 - qd

          with jax.named_scope("p1_writeout"):
            for d in range(3):
                pltpu.sync_copy(gxstage.at[pl.ds(d * QW * K, QW * K)],
                                out_hbm.at[b, d, pl.ds(qoff * K, QW * K)])
            pltpu.sync_copy(idx_all.at[pl.ds(qoff * K, QW * K)],
                            shidx.at[slot, pl.ds(qoff * K, QW * K)])

        pl.run_scoped(
            phase1,
            pltpu.VMEM((3 * N,), jnp.float32),
            pltpu.VMEM((3 * QW,), jnp.float32),
            pltpu.VMEM((K + 8 * _L + _L,), jnp.int32),
            pltpu.VMEM((3 * QW * K,), jnp.float32),
        )

        plsc.subcore_barrier()

        coff = (wid % WPB) * CW
        rsems = [rs0, rs1, rs2, rs3]
        osems = [os0, os1, os2, os3]
        NCH = 16
        OCW = (S * K) // NCH
        NPAIR = CW // 2

        def phase2(r0, r1, r2, r3, oc0, oc1, oc2, oc3):
          with jax.named_scope("p2_group"):
            pltpu.sync_copy(shidx.at[slot], idx_all)
            rowbufs = [r0, r1, r2, r3]
            ochbufs = [oc0, oc1, oc2, oc3]

            def row_dma(ci, buf):
                return pltpu.make_async_copy(
                    feat_hbm.at[b, coff + ci], rowbufs[buf], rsems[buf])

            def out_dma(cc, q, obuf):
                return pltpu.make_async_copy(
                    ochbufs[obuf],
                    out_hbm.at[b, 3 + coff + cc, pl.ds(q * OCW, OCW)],
                    osems[obuf])

            for buf in range(4):
                row_dma(buf, buf).start()

            @pl.loop(0, NPAIR, step=2)
            def _pairs(p0):
                for ps in range(2):
                    p = p0 + ps
                    rb0, rb1 = rowbufs[2 * ps], rowbufs[2 * ps + 1]
                    cc = 2 * p
                    row_dma(cc, 2 * ps).wait()
                    row_dma(cc + 1, 2 * ps + 1).wait()
                    for q in range(NCH):
                        ob0, ob1 = 2 * (q % 2), 2 * (q % 2) + 1
                        gci = p * NCH + q

                        @pl.when(gci >= 2)
                        def _wait_prev():
                            g2 = gci - 2
                            pp, qq = g2 // NCH, g2 % NCH
                            out_dma(2 * pp, qq, ob0).wait()
                            out_dma(2 * pp + 1, qq, ob1).wait()

                        @plsc.parallel_loop(0, OCW, step=_L, unroll=8)
                        def _gather(o):
                            iv = idx_all[pl.ds(q * OCW + o, _L)]
                            ochbufs[ob0][pl.ds(o, _L)] = plsc.load_gather(
                                rb0, [iv])
                            ochbufs[ob1][pl.ds(o, _L)] = plsc.load_gather(
                                rb1, [iv])

                        out_dma(cc, q, ob0).start()
                        out_dma(cc + 1, q, ob1).start()

                    @pl.when(cc + 5 < CW)
                    def _next_rows():
                        row_dma(cc + 4, 2 * ps).start()
                        row_dma(cc + 5, 2 * ps + 1).start()

            for g2 in (NPAIR * NCH - 2, NPAIR * NCH - 1):
                pp, qq = g2 // NCH, g2 % NCH
                out_dma(2 * pp, qq, 2 * (qq % 2)).wait()
                out_dma(2 * pp + 1, qq, 2 * (qq % 2) + 1).wait()

        pl.run_scoped(
            phase2,
            pltpu.VMEM((N,), jnp.float32),
            pltpu.VMEM((N,), jnp.float32),
            pltpu.VMEM((N,), jnp.float32),
            pltpu.VMEM((N,), jnp.float32),
            pltpu.VMEM((OCW,), jnp.float32),
            pltpu.VMEM((OCW,), jnp.float32),
            pltpu.VMEM((OCW,), jnp.float32),
            pltpu.VMEM((OCW,), jnp.float32),
        )

    return qag(xt, nxt, features)


def kernel(xyz, new_xyz, features):
    B, N, _ = xyz.shape
    S = new_xyz.shape[1]
    C = features.shape[1]
    xt = jnp.transpose(xyz, (2, 0, 1)).reshape(3 * B * N)
    nxt = jnp.transpose(new_xyz, (2, 0, 1)).reshape(3 * B * S)
    out = _qag(xt, nxt, features)
    return out.reshape(B, 3 + C, S, _K)


# speedup vs baseline: 3.3007x; 1.0062x over previous
import dataclasses
import functools

import numpy as np
import jax
import jax.numpy as jnp
from jax import lax
from jax.experimental import pallas as pl
from jax.experimental.pallas import tpu as pltpu
from jax.experimental.pallas import tpu_sc as plsc

_RADIUS2 = np.float32(0.2 * 0.2)
_K = 32
_L = 16
_NC = 2
_NS = 16


def _qag(xt, nxt, features):
    B, C, N = features.shape
    S = nxt.shape[0] // (3 * B)
    K = _K
    NW = _NC * _NS
    QW = (B * S) // NW
    WPB = NW // B
    CW = C // WPB
    QTR = (S * K) // 4

    mesh = plsc.VectorSubcoreMesh(core_axis_name="c", subcore_axis_name="s")
    cp = pltpu.CompilerParams()
    if "needs_layout_passes" in pltpu.CompilerParams.__dataclass_fields__:
        cp = dataclasses.replace(cp, needs_layout_passes=False)

    @functools.partial(
        pl.kernel,
        out_type=jax.ShapeDtypeStruct((B, 3 + C, S * K), jnp.float32),
        mesh=mesh,
        compiler_params=cp,
        scratch_types=[
            pltpu.VMEM((S * K,), jnp.int32),
            pltpu.VMEM_SHARED((2, S * K), jnp.int32),
            pltpu.SemaphoreType.DMA,
            pltpu.SemaphoreType.DMA,
            pltpu.SemaphoreType.DMA,
            pltpu.SemaphoreType.DMA,
            pltpu.SemaphoreType.DMA,
            pltpu.SemaphoreType.DMA,
            pltpu.SemaphoreType.DMA,
            pltpu.SemaphoreType.DMA,
        ],
    )
    def qag(xt_hbm, nxt_hbm, feat_hbm, out_hbm,
            idx_all, shidx, rs0, rs1, rs2, rs3, os0, os1, os2, os3):
        iota16 = lax.iota(jnp.int32, _L)

        def _splat(v, lane):
            sel = jnp.where(iota16 == lane, v, jnp.zeros_like(v))
            return jnp.full((_L,), jnp.sum(sel), dtype=v.dtype)

        wid = lax.axis_index("c") * _NS + lax.axis_index("s")
        b = wid // WPB
        slot = b % 2
        qoff = (wid % WPB) * QW

        def phase1(pts, q, idxbuf, gxstage):
          with jax.named_scope("p1_load"):
            for d in range(3):
                pltpu.sync_copy(xt_hbm.at[pl.ds(d * B * N + b * N, N)],
                                pts.at[pl.ds(d * N, N)])
                pltpu.sync_copy(
                    nxt_hbm.at[pl.ds(d * B * S + b * S + qoff, QW)],
                    q.at[pl.ds(d * QW, QW)])

          with jax.named_scope("p1_ballquery"):
            @pl.loop(0, QW)
            def _per_query(qi):
                qiv = jnp.full((_L,), qi, jnp.int32)
                qx = plsc.load_gather(q, [qiv])
                qy = plsc.load_gather(q, [qiv + QW])
                qz = plsc.load_gather(q, [qiv + 2 * QW])
                idxbuf[pl.ds(0, _L)] = jnp.zeros((_L,), jnp.int32)

                def cond(carry):
                    off, cnt = carry
                    return jnp.logical_and(cnt < K, off < N)

                def step(carry):
                    off, cnt = carry
                    cntv = jnp.full((_L,), cnt, jnp.int32)
                    for u in range(8):
                        o = off + u * _L
                        xv = pts[pl.ds(o, _L)]
                        yv = pts[pl.ds(N + o, _L)]
                        zv = pts[pl.ds(2 * N + o, _L)]
                        dx = qx - xv
                        dy = qy - yv
                        dz = qz - zv
                        d2 = dx * dx + dy * dy + dz * dz
                        m = d2 < _RADIUS2
                        pfx = plsc.cumsum(jnp.where(m, 1, 0))
                        plsc.store_scatter(idxbuf, [cntv + (pfx - 1)],
                                           iota16 + o, mask=m)
                        cntv = cntv + plsc.all_reduce_population_count(m)
                    cnt = jnp.sum(jnp.where(iota16 == 0, cntv, 0))
                    return off + 8 * _L, cnt

                _, cnt = lax.while_loop(cond, step,
                                        (jnp.int32(0), jnp.int32(0)))

                k0 = idxbuf[pl.ds(0, _L)]
                k1 = idxbuf[pl.ds(_L, _L)]
                first = plsc.load_gather(idxbuf,
                                         [jnp.zeros((_L,), jnp.int32)])
                cntv = jnp.full((_L,), cnt, jnp.int32)
                f0 = jnp.where(iota16 < cntv, k0, first)
                f1 = jnp.where(iota16 + _L < cntv, k1, first)
                idx_all[pl.ds((qoff + qi) * K, _L)] = f0
                idx_all[pl.ds((qoff + qi) * K + _L, _L)] = f1
                for d in range(3):
                    g0 = plsc.load_gather(pts, [f0 + d * N])
                    g1 = plsc.load_gather(pts, [f1 + d * N])
                    qd = (qx, qy, qz)[d]
                    gxstage[pl.ds(d * QW * K + qi * K, _L)] = g0 - qd
                    gxstage[pl.ds(d * QW * K + qi * K + _L, _L)] = g1 - qd

          with jax.named_scope("p1_writeout"):
            for d in range(3):
                pltpu.sync_copy(gxstage.at[pl.ds(d * QW * K, QW * K)],
                                out_hbm.at[b, d, pl.ds(qoff * K, QW * K)])
            pltpu.sync_copy(idx_all.at[pl.ds(qoff * K, QW * K)],
                            shidx.at[slot, pl.ds(qoff * K, QW * K)])

        pl.run_scoped(
            phase1,
            pltpu.VMEM((3 * N,), jnp.float32),
            pltpu.VMEM((3 * QW,), jnp.float32),
            pltpu.VMEM((K + 8 * _L + _L,), jnp.int32),
            pltpu.VMEM((3 * QW * K,), jnp.float32),
        )

        plsc.subcore_barrier()

        coff = (wid % WPB) * CW
        rsems = [rs0, rs1, rs2, rs3]
        osems = [os0, os1, os2, os3]
        NCH = 16
        OCW = (S * K) // NCH
        NPAIR = CW // 2

        def phase2(r0, r1, r2, r3, oc0, oc1, oc2, oc3):
          with jax.named_scope("p2_group"):
            pltpu.sync_copy(shidx.at[slot], idx_all)
            rowbufs = [r0, r1, r2, r3]
            ochbufs = [oc0, oc1, oc2, oc3]

            def row_dma(ci, buf):
                return pltpu.make_async_copy(
                    feat_hbm.at[b, coff + ci], rowbufs[buf], rsems[buf])

            def out_dma(cc, q, obuf):
                return pltpu.make_async_copy(
                    ochbufs[obuf],
                    out_hbm.at[b, 3 + coff + cc, pl.ds(q * OCW, OCW)],
                    osems[obuf])

            for buf in range(4):
                row_dma(buf, buf).start()

            @pl.loop(0, NPAIR, step=2)
            def _pairs(p0):
                for ps in range(2):
                    p = p0 + ps
                    rb0, rb1 = rowbufs[2 * ps], rowbufs[2 * ps + 1]
                    cc = 2 * p
                    row_dma(cc, 2 * ps).wait()
                    row_dma(cc + 1, 2 * ps + 1).wait()
                    for q in range(NCH):
                        ob0, ob1 = 2 * (q % 2), 2 * (q % 2) + 1
                        gci = p * NCH + q

                        @pl.when(gci >= 2)
                        def _wait_prev():
                            g2 = gci - 2
                            pp, qq = g2 // NCH, g2 % NCH
                            out_dma(2 * pp, qq, ob0).wait()
                            out_dma(2 * pp + 1, qq, ob1).wait()

                        @plsc.parallel_loop(0, OCW, step=_L, unroll=8)
                        def _gather(o):
                            iv = idx_all[pl.ds(q * OCW + o, _L)]
                            ochbufs[ob0][pl.ds(o, _L)] = plsc.load_gather(
                                rb0, [iv])
                            ochbufs[ob1][pl.ds(o, _L)] = plsc.load_gather(
                                rb1, [iv])

                        out_dma(cc, q, ob0).start()
                        out_dma(cc + 1, q, ob1).start()

                    @pl.when(cc + 5 < CW)
                    def _next_rows():
                        row_dma(cc + 4, 2 * ps).start()
                        row_dma(cc + 5, 2 * ps + 1).start()

            for g2 in (NPAIR * NCH - 2, NPAIR * NCH - 1):
                pp, qq = g2 // NCH, g2 % NCH
                out_dma(2 * pp, qq, 2 * (qq % 2)).wait()
                out_dma(2 * pp + 1, qq, 2 * (qq % 2) + 1).wait()

        pl.run_scoped(
            phase2,
            pltpu.VMEM((N,), jnp.float32),
            pltpu.VMEM((N,), jnp.float32),
            pltpu.VMEM((N,), jnp.float32),
            pltpu.VMEM((N,), jnp.float32),
            pltpu.VMEM((OCW,), jnp.float32),
            pltpu.VMEM((OCW,), jnp.float32),
            pltpu.VMEM((OCW,), jnp.float32),
            pltpu.VMEM((OCW,), jnp.float32),
        )

    return qag(xt, nxt, features)


def kernel(xyz, new_xyz, features):
    B, N, _ = xyz.shape
    S = new_xyz.shape[1]
    C = features.shape[1]
    xt = jnp.transpose(xyz, (2, 0, 1)).reshape(3 * B * N)
    nxt = jnp.transpose(new_xyz, (2, 0, 1)).reshape(3 * B * S)
    out = _qag(xt, nxt, features)
    return out.reshape(B, 3 + C, S, _K)
